# baseline (speedup 1.0000x reference)
import functools

import jax
import jax.numpy as jnp
from jax.experimental import pallas as pl
from jax.experimental.pallas import tpu as pltpu

K = 8
GLOBAL_COEFF = 0.5
EPS = 1e-8


def _dot3(a, b_hi, b_lo, dims):
    a_hi = a.astype(jnp.bfloat16)
    a_lo = (a - a_hi.astype(jnp.float32)).astype(jnp.bfloat16)
    d = functools.partial(jax.lax.dot_general, dimension_numbers=dims,
                          preferred_element_type=jnp.float32)
    return d(a_hi, b_hi) + (d(a_hi, b_lo) + d(a_lo, b_hi))


NSPLIT = 2


def _fused_body(nsteps, P, ppg_ref, *refs):
    k_refs = refs[:NSPLIT]
    p_refs = refs[NSPLIT:2 * NSPLIT]
    out_ref, ssum_ref, ent_ref, score_ref, pscr_ref = refs[2 * NSPLIT:]
    i = pl.program_id(0)
    tile_p = k_refs[0].shape[0]
    dims = (((1,), (1,)), ((), ()))

    ppg = ppg_ref[...].reshape(ppg_ref.shape[0], ppg_ref.shape[2])
    na = jnp.sqrt(jnp.sum(ppg * ppg, axis=1, keepdims=True))

    for part, (k_ref, p_ref) in enumerate(zip(k_refs, p_refs)):
        keys = k_ref[...]
        k_hi = keys.astype(jnp.bfloat16)
        k_lo = (keys - k_hi.astype(jnp.float32)).astype(jnp.bfloat16)
        dot = _dot3(ppg, k_hi, k_lo, dims)

        nb = jnp.sqrt(jnp.sum(keys * keys, axis=1, keepdims=True)).T
        denom = jnp.maximum(na, EPS) * jnp.maximum(nb, EPS)
        off = (NSPLIT * i + part) * tile_p
        score_ref[:, pl.ds(off, tile_p)] = 1.0 - dot / denom
        pscr_ref[pl.ds(off, tile_p), :] = p_ref[...].astype(jnp.bfloat16)

    @pl.when(i == nsteps - 1)
    def _finalize():
        score = score_ref[...]
        bz = score.shape[0]
        iota = jax.lax.broadcasted_iota(jnp.int32, (bz, P), 1)

        mx = jnp.max(score, axis=1, keepdims=True)
        ex = jnp.exp(score - mx)
        se = jnp.sum(ex, axis=1, keepdims=True)
        logp = (score - mx) - jnp.log(se)
        ent_ref[0, 0] = -jnp.sum((ex / se) * logp)

        work = score
        mask = jnp.zeros_like(score)
        for _ in range(K):
            mv = jnp.min(work, axis=1, keepdims=True)
            first = jnp.min(jnp.where(work == mv, iota, P), axis=1,
                            keepdims=True)
            sel = iota == first
            mask = jnp.where(sel, 1.0, mask)
            work = jnp.where(sel, jnp.inf, work)
        ssum_ref[0, 0] = jnp.sum(score * mask)

        psum = jax.lax.dot_general(
            mask.astype(jnp.bfloat16), pscr_ref[...], (((1,), (0,)), ((), ())),
            preferred_element_type=jnp.float32)
        res = ppg + GLOBAL_COEFF * psum
        out_ref[...] = res.reshape(out_ref.shape)


def kernel(ppg, keys, prompt):
    bz, _, d = ppg.shape
    p = keys.shape[0]

    tile_p = 128
    nsteps = p // (NSPLIT * tile_p)

    def _map(part):
        return lambda i: (NSPLIT * i + part, 0)

    stream_specs = [pl.BlockSpec((tile_p, d), _map(part))
                    for part in range(NSPLIT)]
    prompted, ssum, ent = pl.pallas_call(
        functools.partial(_fused_body, nsteps, p),
        grid=(nsteps,),
        in_specs=[pl.BlockSpec((bz, 1, d), lambda i: (0, 0, 0))]
                 + stream_specs + stream_specs,
        out_specs=[
            pl.BlockSpec((bz, 1, d), lambda i: (0, 0, 0)),
            pl.BlockSpec(memory_space=pltpu.SMEM),
            pl.BlockSpec(memory_space=pltpu.SMEM),
        ],
        out_shape=[
            jax.ShapeDtypeStruct((bz, 1, d), jnp.float32),
            jax.ShapeDtypeStruct((1, 1), jnp.float32),
            jax.ShapeDtypeStruct((1, 1), jnp.float32),
        ],
        scratch_shapes=[
            pltpu.VMEM((bz, p), jnp.float32),
            pltpu.VMEM((p, d), jnp.bfloat16),
        ],
    )(ppg, *([keys] * NSPLIT), *([prompt] * NSPLIT))

    return prompted, ssum[0, 0], ent[0, 0]


# speedup vs baseline: 1.5443x; 1.5443x over previous
import functools

import jax
import jax.numpy as jnp
from jax.experimental import pallas as pl
from jax.experimental.pallas import tpu as pltpu

K = 8
GLOBAL_COEFF = 0.5
EPS = 1e-8


def _dot3(a, b_hi, b_lo, dims):
    a_hi = a.astype(jnp.bfloat16)
    a_lo = (a - a_hi.astype(jnp.float32)).astype(jnp.bfloat16)
    d = functools.partial(jax.lax.dot_general, dimension_numbers=dims,
                          preferred_element_type=jnp.float32)
    return d(a_hi, b_hi) + (d(a_hi, b_lo) + d(a_lo, b_hi))


NSPLIT = 2


def _fused_body(nsteps, P, ppg_ref, *refs):
    k_refs = refs[:NSPLIT]
    p_refs = refs[NSPLIT:2 * NSPLIT]
    out_ref, ssum_ref, ent_ref, score_ref, pscr_ref = refs[2 * NSPLIT:]
    i = pl.program_id(0)
    tile_p = k_refs[0].shape[0]
    dims = (((1,), (1,)), ((), ()))

    ppg = ppg_ref[...]
    na = jnp.sqrt(jnp.sum(ppg * ppg, axis=1, keepdims=True))

    for part, (k_ref, p_ref) in enumerate(zip(k_refs, p_refs)):
        keys = k_ref[...]
        k_hi = keys.astype(jnp.bfloat16)
        k_lo = (keys - k_hi.astype(jnp.float32)).astype(jnp.bfloat16)
        dot = _dot3(ppg, k_hi, k_lo, dims)

        nb = jnp.sqrt(jnp.sum(keys * keys, axis=1, keepdims=True)).T
        denom = jnp.maximum(na, EPS) * jnp.maximum(nb, EPS)
        off = (NSPLIT * i + part) * tile_p
        score_ref[:, pl.ds(off, tile_p)] = 1.0 - dot / denom
        pscr_ref[pl.ds(off, tile_p), :] = p_ref[...].astype(jnp.bfloat16)

    @pl.when(i == nsteps - 1)
    def _finalize():
        score = score_ref[...]
        bz = score.shape[0]
        iota = jax.lax.broadcasted_iota(jnp.int32, (bz, P), 1)

        mx = jnp.max(score, axis=1, keepdims=True)
        ex = jnp.exp(score - mx)
        se = jnp.sum(ex, axis=1, keepdims=True)
        logp = (score - mx) - jnp.log(se)
        ent_ref[0, 0] = -jnp.sum((ex / se) * logp)

        work = score
        mask = jnp.zeros_like(score)
        for _ in range(K):
            mv = jnp.min(work, axis=1, keepdims=True)
            first = jnp.min(jnp.where(work == mv, iota, P), axis=1,
                            keepdims=True)
            sel = iota == first
            mask = jnp.where(sel, 1.0, mask)
            work = jnp.where(sel, jnp.inf, work)
        ssum_ref[0, 0] = jnp.sum(score * mask)

        psum = jax.lax.dot_general(
            mask.astype(jnp.bfloat16), pscr_ref[...], (((1,), (0,)), ((), ())),
            preferred_element_type=jnp.float32)
        out_ref[...] = ppg + GLOBAL_COEFF * psum


def kernel(ppg, keys, prompt):
    bz, _, d = ppg.shape
    p = keys.shape[0]
    ppg2d = ppg.reshape(bz, d)

    tile_p = 128
    nsteps = p // (NSPLIT * tile_p)

    def _map(part):
        return lambda i: (NSPLIT * i + part, 0)

    stream_specs = [pl.BlockSpec((tile_p, d), _map(part))
                    for part in range(NSPLIT)]
    prompted, ssum, ent = pl.pallas_call(
        functools.partial(_fused_body, nsteps, p),
        grid=(nsteps,),
        in_specs=[pl.BlockSpec((bz, d), lambda i: (0, 0))]
                 + stream_specs + stream_specs,
        out_specs=[
            pl.BlockSpec((bz, d), lambda i: (0, 0)),
            pl.BlockSpec(memory_space=pltpu.SMEM),
            pl.BlockSpec(memory_space=pltpu.SMEM),
        ],
        out_shape=[
            jax.ShapeDtypeStruct((bz, d), jnp.float32),
            jax.ShapeDtypeStruct((1, 1), jnp.float32),
            jax.ShapeDtypeStruct((1, 1), jnp.float32),
        ],
        scratch_shapes=[
            pltpu.VMEM((bz, p), jnp.float32),
            pltpu.VMEM((p, d), jnp.bfloat16),
        ],
    )(ppg2d, *([keys] * NSPLIT), *([prompt] * NSPLIT))

    return prompted.reshape(bz, 1, d), ssum[0, 0], ent[0, 0]
